# baseline (speedup 1.0000x reference)
import functools

import jax
import jax.numpy as jnp
from jax import lax
from jax.experimental import pallas as pl
from jax.experimental.pallas import tpu as pltpu
from jax.experimental.pallas import tpu_sc as plsc

_BATCH = 16384
_FIELDS = 100
_LANES = 16
_NUM_WORKERS = 32
_COLS_W = _BATCH // _NUM_WORKERS
_SEG = 128
_NSEG = _COLS_W // _SEG
_UNROLL = _SEG // _LANES

_mesh = plsc.VectorSubcoreMesh(core_axis_name="c", subcore_axis_name="s")


@functools.partial(
    pl.kernel,
    mesh=_mesh,
    out_type=jax.ShapeDtypeStruct((_NUM_WORKERS, _LANES), jnp.float32),
    compiler_params=pltpu.CompilerParams(use_tc_tiling_on_sc=True),
    scratch_types=[
        pltpu.VMEM((_FIELDS, _NSEG, _SEG), jnp.int32),
        pltpu.VMEM((_FIELDS, _NSEG, _SEG), jnp.float32),
        pltpu.VMEM((_LANES,), jnp.float32),
        pltpu.SemaphoreType.DMA,
    ] + [pltpu.SemaphoreType.DMA] * _NSEG,
)
def _gather_sum(idx_hbm, w_hbm, out_hbm, idx_v, vals_v, acc_v, gsem, *ssems):
    nc = plsc.get_sparse_core_info().num_cores
    wid = lax.axis_index("s") * nc + lax.axis_index("c")
    col0 = wid * _COLS_W

    stage_h = [
        pltpu.async_copy(
            idx_hbm.at[:, pl.ds(col0 + k * _SEG, _SEG)], idx_v.at[:, k, :],
            ssems[k])
        for k in range(_NSEG)
    ]

    for k in range(_NSEG):
        stage_h[k].wait()

        def issue(r, carry):
            pltpu.async_copy(w_hbm.at[idx_v.at[r, k]], vals_v.at[r, k], gsem)
            return carry

        lax.fori_loop(0, _FIELDS, issue, 0)

    for k in range(_NSEG):

        def drain(r, carry):
            pltpu.make_async_copy(
                w_hbm.at[idx_v.at[r, k]], vals_v.at[r, k], gsem).wait()
            return carry

        lax.fori_loop(0, _FIELDS, drain, 0)

    def body(r, accs):
        new = list(accs)
        for k in range(_NSEG):
            for j in range(_UNROLL):
                new[j] = new[j] + vals_v[r, k, pl.ds(j * _LANES, _LANES)]
        return tuple(new)

    zeros = jnp.zeros((_LANES,), jnp.float32)
    accs = lax.fori_loop(0, _FIELDS, body, (zeros,) * _UNROLL)
    total = accs[0]
    for j in range(1, _UNROLL):
        total = total + accs[j]
    acc_v[...] = total
    pltpu.sync_copy(acc_v, out_hbm.at[wid])


def kernel(non_zero_indices, weights):
    partials = _gather_sum(non_zero_indices.T, weights)
    return jnp.sum(partials)


# speedup vs baseline: 1.0188x; 1.0188x over previous
import functools

import jax
import jax.numpy as jnp
from jax import lax
from jax.experimental import pallas as pl
from jax.experimental.pallas import tpu as pltpu
from jax.experimental.pallas import tpu_sc as plsc

_BATCH = 16384
_FIELDS = 100
_LANES = 16
_NUM_WORKERS = 32
_COLS_W = _BATCH // _NUM_WORKERS
_SEG = 128
_NSEG = _COLS_W // _SEG
_UNROLL = _SEG // _LANES

_mesh = plsc.VectorSubcoreMesh(core_axis_name="c", subcore_axis_name="s")


@functools.partial(
    pl.kernel,
    mesh=_mesh,
    out_type=jax.ShapeDtypeStruct((_NUM_WORKERS, _LANES), jnp.float32),
    compiler_params=pltpu.CompilerParams(use_tc_tiling_on_sc=True),
    scratch_types=[
        pltpu.VMEM((_FIELDS, _NSEG, _SEG), jnp.int32),
        pltpu.VMEM((_FIELDS, _NSEG, _SEG), jnp.float32),
        pltpu.VMEM((_LANES,), jnp.float32),
    ] + [pltpu.SemaphoreType.DMA] * (2 * _NSEG),
)
def _gather_sum(idx_hbm, w_hbm, out_hbm, idx_v, vals_v, acc_v, *sems):
    ssems = sems[:_NSEG]
    gsems = sems[_NSEG:]
    nc = plsc.get_sparse_core_info().num_cores
    wid = lax.axis_index("s") * nc + lax.axis_index("c")
    col0 = wid * _COLS_W

    stage_h = [
        pltpu.async_copy(
            idx_hbm.at[:, pl.ds(col0 + k * _SEG, _SEG)], idx_v.at[:, k, :],
            ssems[k])
        for k in range(_NSEG)
    ]

    for k in range(_NSEG):
        stage_h[k].wait()

        def issue(r, carry):
            pltpu.async_copy(
                w_hbm.at[idx_v.at[r, k]], vals_v.at[r, k], gsems[k])
            return carry

        lax.fori_loop(0, _FIELDS, issue, 0)

    zeros = jnp.zeros((_LANES,), jnp.float32)
    accs = (zeros,) * _UNROLL
    for k in range(_NSEG):

        def drain(r, carry):
            pltpu.make_async_copy(
                w_hbm.at[idx_v.at[r, k]], vals_v.at[r, k], gsems[k]).wait()
            return carry

        lax.fori_loop(0, _FIELDS, drain, 0)

        def body(r, a):
            return tuple(
                a[j] + vals_v[r, k, pl.ds(j * _LANES, _LANES)]
                for j in range(_UNROLL)
            )

        accs = lax.fori_loop(0, _FIELDS, body, accs)
    total = accs[0]
    for j in range(1, _UNROLL):
        total = total + accs[j]
    acc_v[...] = total
    pltpu.sync_copy(acc_v, out_hbm.at[wid])


def kernel(non_zero_indices, weights):
    partials = _gather_sum(non_zero_indices.T, weights)
    return jnp.sum(partials)
